# Initial kernel scaffold; baseline (speedup 1.0000x reference)
import jax
import jax.numpy as jnp
from jax.experimental import pallas as pl


def kernel(x, edge_index, h_prev1, h_prev2, W_gat, attn_l, attn_r, b_gat, Wz1, bz1, Wh1, bh1, Wz2, bz2, Wh2, bh2, We1, be1, We2, be2, We3, be3, Wf, bf):
    raise NotImplementedError("write your pallas kernel here")


# speedup vs baseline: 21.1904x; 21.1904x over previous
import functools

import jax
import jax.numpy as jnp
from jax import lax
from jax.experimental import pallas as pl
from jax.experimental.pallas import tpu as pltpu
from jax.experimental.pallas import tpu_sc as plsc

N = 10000
E = 320000
D_IN = 128
HID = 128
EMB = 16

NC = 2
NS = 16
NW = NC * NS
EPW = E // NW
CHUNK = 80
NCH = EPW // CHUNK
NP = 10240
RPT = NP // NS
ZR = 128
DCH = 2048


def _pre_body(x_ref, wg_ref, al_ref, ar_ref, h_ref, a1_ref, a2_ref):
    h = jnp.dot(x_ref[...], wg_ref[...], preferred_element_type=jnp.float32)
    h_ref[...] = h
    a1_ref[...] = jnp.sum(h * al_ref[...], axis=1, keepdims=True)
    a2_ref[...] = jnp.sum(h * ar_ref[...], axis=1, keepdims=True)


def _pre_call(x, wg, al, ar):
    return pl.pallas_call(
        _pre_body,
        out_shape=[
            jax.ShapeDtypeStruct((N, HID), jnp.float32),
            jax.ShapeDtypeStruct((N, 1), jnp.float32),
            jax.ShapeDtypeStruct((N, 1), jnp.float32),
        ],
    )(x, wg, al, ar)


_sc_mesh = plsc.VectorSubcoreMesh(core_axis_name="c", subcore_axis_name="s")


@functools.partial(
    pl.kernel,
    out_type=(
        jax.ShapeDtypeStruct((NC, NP, HID), jnp.float32),
        jax.ShapeDtypeStruct((NC, NP), jnp.float32),
    ),
    mesh=_sc_mesh,
    scratch_types=[
        pltpu.VMEM((N,), jnp.float32),
        pltpu.VMEM((N,), jnp.float32),
        pltpu.VMEM((1, CHUNK), jnp.int32),
        pltpu.VMEM((1, CHUNK), jnp.int32),
        pltpu.VMEM((1, CHUNK), jnp.float32),
        pltpu.VMEM((CHUNK, HID), jnp.float32),
        pltpu.VMEM((DCH,), jnp.float32),
        pltpu.VMEM_SHARED((NP, HID), jnp.float32),
        pltpu.VMEM_SHARED((NP,), jnp.float32),
        pltpu.SemaphoreType.DMA,
    ],
    compiler_params=pltpu.CompilerParams(needs_layout_passes=False),
)
def _sc_edge(src_hbm, dst_hbm, a1_hbm, a2_hbm, h_hbm, num_out, den_out,
             a1_v, a2_v, srcc_v, dstc_v, wc_v, rows_v, zden_v,
             num_sh, den_sh, sem):
    c = lax.axis_index("c")
    s = lax.axis_index("s")
    wid = c * NS + s

    pltpu.sync_copy(a1_hbm, a1_v)
    pltpu.sync_copy(a2_hbm, a2_v)

    zf = jnp.zeros((16,), jnp.float32)

    def _zrow(k, _):
        for j in range(HID // 16):
            rows_v[k, pl.ds(j * 16, 16)] = zf
        return 0

    lax.fori_loop(0, CHUNK, _zrow, 0)

    def _zden(k, _):
        zden_v[pl.ds(k * 16, 16)] = zf
        return 0

    lax.fori_loop(0, DCH // 16, _zden, 0)

    for t in range(RPT // CHUNK):
        pltpu.sync_copy(rows_v, num_sh.at[pl.ds(s * RPT + t * CHUNK, CHUNK)])

    @pl.when(s < NP // DCH)
    def _():
        pltpu.sync_copy(zden_v, den_sh.at[pl.ds(s * DCH, DCH)])

    plsc.subcore_barrier()

    def _chunk(i, _):
        pltpu.sync_copy(src_hbm.at[wid, pl.ds(i, 1)], srcc_v)
        pltpu.sync_copy(dst_hbm.at[wid, pl.ds(i, 1)], dstc_v)
        pltpu.async_copy(h_hbm.at[srcc_v.at[0]], rows_v, sem).wait()

        def _wgrp(j, _):
            s16 = srcc_v[0, pl.ds(j * 16, 16)]
            d16 = dstc_v[0, pl.ds(j * 16, 16)]
            e = plsc.load_gather(a1_v, [s16]) + plsc.load_gather(a2_v, [d16])
            e = jnp.where(e >= 0, e, 0.2 * e)
            wc_v[0, pl.ds(j * 16, 16)] = jnp.exp(e)
            return 0

        lax.fori_loop(0, CHUNK // 16, _wgrp, 0)

        def _scale(k, _):
            wk = plsc.load_gather(wc_v.at[0], [jnp.full((16,), k, jnp.int32)])
            for j in range(HID // 16):
                rows_v[k, pl.ds(j * 16, 16)] = rows_v[k, pl.ds(j * 16, 16)] * wk
            return 0

        lax.fori_loop(0, CHUNK, _scale, 0)

        pltpu.sync_copy(rows_v, num_sh.at[dstc_v.at[0]], add=True)
        pltpu.sync_copy(wc_v.at[0], den_sh.at[dstc_v.at[0]], add=True)
        return 0

    lax.fori_loop(0, NCH, _chunk, 0)

    plsc.subcore_barrier()

    pltpu.sync_copy(num_sh.at[pl.ds(s * RPT, RPT)],
                    num_out.at[c, pl.ds(s * RPT, RPT)])

    @pl.when(s < NP // DCH)
    def _():
        pltpu.sync_copy(den_sh.at[pl.ds(s * DCH, DCH)],
                        den_out.at[c, pl.ds(s * DCH, DCH)])


BLK = 1000


def _post_body(num_ref, den_ref, hp1_ref, hp2_ref, bg_ref,
               wz1_ref, bz1_ref, wh1_ref, bh1_ref,
               wz2_ref, bz2_ref, wh2_ref, bh2_ref,
               we_ref, be_ref, wf_ref, bf_ref, c_ref, tp_ref):
    num = num_ref[0] + num_ref[1]
    den = den_ref[0, :, 0] + den_ref[1, :, 0]
    out = num / (den[:, None] + 1e-16) + bg_ref[...]
    z1 = jax.nn.sigmoid(
        jnp.dot(out, wz1_ref[...], preferred_element_type=jnp.float32)
        + bz1_ref[...])
    ht1 = jnp.tanh(
        jnp.dot(out, wh1_ref[...], preferred_element_type=jnp.float32)
        + bh1_ref[...])
    h1 = (1.0 - z1) * hp1_ref[...] + z1 * ht1
    z2 = jax.nn.sigmoid(
        jnp.dot(h1, wz2_ref[...], preferred_element_type=jnp.float32)
        + bz2_ref[...])
    ht2 = jnp.tanh(
        jnp.dot(h1, wh2_ref[...], preferred_element_type=jnp.float32)
        + bh2_ref[...])
    h2 = (1.0 - z2) * hp2_ref[...] + z2 * ht2
    c_ref[...] = (jnp.dot(h2, we_ref[...], preferred_element_type=jnp.float32)
                  + be_ref[...])
    tp_ref[...] = (jnp.dot(h2, wf_ref[...], preferred_element_type=jnp.float32)
                   + bf_ref[...])


def _post_call(num, den3, hp1, hp2, bg, wz1, bz1, wh1, bh1,
               wz2, bz2, wh2, bh2, we, be, wf, bf):
    full = lambda shape: pl.BlockSpec(shape, lambda i: (0,) * len(shape))
    return pl.pallas_call(
        _post_body,
        grid=(N // BLK,),
        in_specs=[
            pl.BlockSpec((NC, BLK, HID), lambda i: (0, i, 0)),
            pl.BlockSpec((NC, BLK, 1), lambda i: (0, i, 0)),
            pl.BlockSpec((BLK, HID), lambda i: (i, 0)),
            pl.BlockSpec((BLK, HID), lambda i: (i, 0)),
            full((1, HID)),
            full((HID, HID)), full((1, HID)),
            full((HID, HID)), full((1, HID)),
            full((HID, HID)), full((1, HID)),
            full((HID, HID)), full((1, HID)),
            full((HID, 3 * EMB)), full((1, 3 * EMB)),
            full((HID, 1)), full((1, 1)),
        ],
        out_specs=[
            pl.BlockSpec((BLK, 3 * EMB), lambda i: (i, 0)),
            pl.BlockSpec((BLK, 1), lambda i: (i, 0)),
        ],
        out_shape=[
            jax.ShapeDtypeStruct((N, 3 * EMB), jnp.float32),
            jax.ShapeDtypeStruct((N, 1), jnp.float32),
        ],
    )(num, den3, hp1, hp2, bg, wz1, bz1, wh1, bh1,
      wz2, bz2, wh2, bh2, we, be, wf, bf)


def kernel(x, edge_index, h_prev1, h_prev2, W_gat, attn_l, attn_r, b_gat,
           Wz1, bz1, Wh1, bh1, Wz2, bz2, Wh2, bh2,
           We1, be1, We2, be2, We3, be3, Wf, bf):
    al = attn_l.reshape(1, HID)
    ar = attn_r.reshape(1, HID)
    h, a1k, a2k = _pre_call(x, W_gat, al, ar)

    src = edge_index[0].reshape(NW, NCH, CHUNK)
    dst = edge_index[1].reshape(NW, NCH, CHUNK)
    num, den = _sc_edge(src, dst, a1k.reshape(N), a2k.reshape(N), h)

    we = jnp.concatenate([We1, We2, We3], axis=1)
    be = jnp.concatenate([be1, be2, be3]).reshape(1, 3 * EMB)
    cat, tp = _post_call(
        num, den.reshape(NC, NP, 1), h_prev1, h_prev2, b_gat.reshape(1, HID),
        Wz1, bz1.reshape(1, HID), Wh1, bh1.reshape(1, HID),
        Wz2, bz2.reshape(1, HID), Wh2, bh2.reshape(1, HID),
        we, be, Wf, bf.reshape(1, 1))
    return (cat.reshape(N, 3, EMB), tp.reshape(N))


# speedup vs baseline: 38.6915x; 1.8259x over previous
import functools

import jax
import jax.numpy as jnp
from jax import lax
from jax.experimental import pallas as pl
from jax.experimental.pallas import tpu as pltpu
from jax.experimental.pallas import tpu_sc as plsc

N = 10000
E = 320000
D_IN = 128
HID = 128
EMB = 16

NC = 2
NS = 16
NW = NC * NS
EPW = E // NW
CHUNK = 80
NCH = EPW // CHUNK
NP = 10240
RPT = NP // NS
DCH = 2048


def _pre_body(x_ref, wg_ref, al_ref, ar_ref, h_ref, a1_ref, a2_ref):
    h = jnp.dot(x_ref[...], wg_ref[...], preferred_element_type=jnp.float32)
    h_ref[...] = h
    a1_ref[...] = jnp.sum(h * al_ref[...], axis=1, keepdims=True)
    a2_ref[...] = jnp.sum(h * ar_ref[...], axis=1, keepdims=True)


def _pre_call(x, wg, al, ar):
    return pl.pallas_call(
        _pre_body,
        out_shape=[
            jax.ShapeDtypeStruct((N, HID), jnp.float32),
            jax.ShapeDtypeStruct((N, 1), jnp.float32),
            jax.ShapeDtypeStruct((N, 1), jnp.float32),
        ],
    )(x, wg, al, ar)


_sc_mesh = plsc.VectorSubcoreMesh(core_axis_name="c", subcore_axis_name="s")


@functools.partial(
    pl.kernel,
    out_type=(
        jax.ShapeDtypeStruct((NC, NP, HID), jnp.float32),
        jax.ShapeDtypeStruct((NC, NP), jnp.float32),
    ),
    mesh=_sc_mesh,
    scratch_types=[
        pltpu.VMEM((NP,), jnp.float32),
        pltpu.VMEM((N,), jnp.float32),
        pltpu.VMEM((2, CHUNK), jnp.int32),
        pltpu.VMEM((2, CHUNK), jnp.int32),
        pltpu.VMEM((1, CHUNK), jnp.float32),
        pltpu.VMEM((CHUNK, HID), jnp.float32),
        pltpu.VMEM((CHUNK, HID), jnp.float32),
        pltpu.VMEM_SHARED((NP, HID), jnp.float32),
        pltpu.VMEM_SHARED((NP,), jnp.float32),
        pltpu.SemaphoreType.DMA,
        pltpu.SemaphoreType.DMA,
        pltpu.SemaphoreType.DMA,
        pltpu.SemaphoreType.DMA,
    ],
    compiler_params=pltpu.CompilerParams(needs_layout_passes=False),
)
def _sc_edge(sd_hbm, a1_hbm, a2_hbm, h_hbm, num_out, den_out,
             a1_v, a2_v, sdc0, sdc1, wc_v, rows0, rows1,
             num_sh, den_sh, gsem0, gsem1, isem0, isem1):
    c = lax.axis_index("c")
    s = lax.axis_index("s")
    wid = c * NS + s
    zf = jnp.zeros((16,), jnp.float32)

    def _zrow(k, _):
        for j in range(HID // 16):
            rows0[k, pl.ds(j * 16, 16)] = zf
        return 0

    lax.fori_loop(0, CHUNK, _zrow, 0)

    def _za(k, _):
        a1_v[pl.ds(k * 16, 16)] = zf
        return 0

    lax.fori_loop(0, NP // 16, _za, 0)

    for t in range(RPT // CHUNK):
        pltpu.sync_copy(rows0, num_sh.at[pl.ds(s * RPT + t * CHUNK, CHUNK)])

    @pl.when(s < NP // DCH)
    def _():
        pltpu.sync_copy(a1_v.at[pl.ds(0, DCH)], den_sh.at[pl.ds(s * DCH, DCH)])

    pltpu.sync_copy(a1_hbm, a1_v.at[pl.ds(0, N)])
    pltpu.sync_copy(a2_hbm, a2_v)

    bufs = ((sdc0, rows0, gsem0, isem0), (sdc1, rows1, gsem1, isem1))

    def _idx_start(ii, b):
        sdc, _, _, isem = bufs[b]
        pltpu.async_copy(sd_hbm.at[wid, ii], sdc, isem)

    def _idx_wait(b):
        sdc, _, _, isem = bufs[b]
        pltpu.make_async_copy(sd_hbm.at[wid, 0], sdc, isem).wait()

    def _gather_start(b):
        sdc, rows, gsem, _ = bufs[b]
        pltpu.async_copy(h_hbm.at[sdc.at[0]], rows, gsem)

    def _gather_wait(b):
        sdc, rows, gsem, _ = bufs[b]
        pltpu.make_async_copy(h_hbm.at[sdc.at[0]], rows, gsem).wait()

    def _process(ii, b, steady):
        sdc, rows, _, _ = bufs[b]
        nb = 1 - b
        if steady:
            _idx_wait(nb)
            _gather_start(nb)

        for j in range(CHUNK // 16):
            s16 = sdc[0, pl.ds(j * 16, 16)]
            d16 = sdc[1, pl.ds(j * 16, 16)]
            e = plsc.load_gather(a1_v, [s16]) + plsc.load_gather(a2_v, [d16])
            e = jnp.where(e >= 0, e, 0.2 * e)
            wc_v[0, pl.ds(j * 16, 16)] = jnp.exp(e)

        _gather_wait(b)

        @plsc.parallel_loop(0, CHUNK, unroll=2)
        def _scale(k):
            wk = plsc.load_gather(wc_v.at[0], [jnp.full((16,), k, jnp.int32)])
            for j in range(HID // 16):
                rows[k, pl.ds(j * 16, 16)] = rows[k, pl.ds(j * 16, 16)] * wk

        pltpu.sync_copy(rows, num_sh.at[sdc.at[1]], add=True)
        pltpu.sync_copy(wc_v.at[0], den_sh.at[sdc.at[1]], add=True)

        if steady:
            @pl.when(ii < NCH - 2)
            def _():
                _idx_start(ii + 2, b)

    _idx_start(0, 0)
    _idx_wait(0)
    _gather_start(0)
    _idx_start(1, 1)

    plsc.subcore_barrier()

    def _pair(t, _):
        _process(2 * t, 0, True)
        _process(2 * t + 1, 1, True)
        return 0

    lax.fori_loop(0, (NCH - 1) // 2, _pair, 0)
    _process(NCH - 1, 0, False)

    plsc.subcore_barrier()

    pltpu.sync_copy(num_sh.at[pl.ds(s * RPT, RPT)],
                    num_out.at[c, pl.ds(s * RPT, RPT)])

    @pl.when(s < NP // DCH)
    def _():
        pltpu.sync_copy(den_sh.at[pl.ds(s * DCH, DCH)],
                        den_out.at[c, pl.ds(s * DCH, DCH)])


BLK = 1000


def _post_body(num_ref, den_ref, hp1_ref, hp2_ref, bg_ref,
               wz1_ref, bz1_ref, wh1_ref, bh1_ref,
               wz2_ref, bz2_ref, wh2_ref, bh2_ref,
               we_ref, be_ref, wf_ref, bf_ref, c_ref, tp_ref):
    num = num_ref[0] + num_ref[1]
    den = den_ref[0, :, 0] + den_ref[1, :, 0]
    out = num / (den[:, None] + 1e-16) + bg_ref[...]
    z1 = jax.nn.sigmoid(
        jnp.dot(out, wz1_ref[...], preferred_element_type=jnp.float32)
        + bz1_ref[...])
    ht1 = jnp.tanh(
        jnp.dot(out, wh1_ref[...], preferred_element_type=jnp.float32)
        + bh1_ref[...])
    h1 = (1.0 - z1) * hp1_ref[...] + z1 * ht1
    z2 = jax.nn.sigmoid(
        jnp.dot(h1, wz2_ref[...], preferred_element_type=jnp.float32)
        + bz2_ref[...])
    ht2 = jnp.tanh(
        jnp.dot(h1, wh2_ref[...], preferred_element_type=jnp.float32)
        + bh2_ref[...])
    h2 = (1.0 - z2) * hp2_ref[...] + z2 * ht2
    c_ref[...] = (jnp.dot(h2, we_ref[...], preferred_element_type=jnp.float32)
                  + be_ref[...])
    tp_ref[...] = (jnp.dot(h2, wf_ref[...], preferred_element_type=jnp.float32)
                   + bf_ref[...])


def _post_call(num, den3, hp1, hp2, bg, wz1, bz1, wh1, bh1,
               wz2, bz2, wh2, bh2, we, be, wf, bf):
    full = lambda shape: pl.BlockSpec(shape, lambda i: (0,) * len(shape))
    return pl.pallas_call(
        _post_body,
        grid=(N // BLK,),
        in_specs=[
            pl.BlockSpec((NC, BLK, HID), lambda i: (0, i, 0)),
            pl.BlockSpec((NC, BLK, 1), lambda i: (0, i, 0)),
            pl.BlockSpec((BLK, HID), lambda i: (i, 0)),
            pl.BlockSpec((BLK, HID), lambda i: (i, 0)),
            full((1, HID)),
            full((HID, HID)), full((1, HID)),
            full((HID, HID)), full((1, HID)),
            full((HID, HID)), full((1, HID)),
            full((HID, HID)), full((1, HID)),
            full((HID, 3 * EMB)), full((1, 3 * EMB)),
            full((HID, 1)), full((1, 1)),
        ],
        out_specs=[
            pl.BlockSpec((BLK, 3 * EMB), lambda i: (i, 0)),
            pl.BlockSpec((BLK, 1), lambda i: (i, 0)),
        ],
        out_shape=[
            jax.ShapeDtypeStruct((N, 3 * EMB), jnp.float32),
            jax.ShapeDtypeStruct((N, 1), jnp.float32),
        ],
    )(num, den3, hp1, hp2, bg, wz1, bz1, wh1, bh1,
      wz2, bz2, wh2, bh2, we, be, wf, bf)


def kernel(x, edge_index, h_prev1, h_prev2, W_gat, attn_l, attn_r, b_gat,
           Wz1, bz1, Wh1, bh1, Wz2, bz2, Wh2, bh2,
           We1, be1, We2, be2, We3, be3, Wf, bf):
    al = attn_l.reshape(1, HID)
    ar = attn_r.reshape(1, HID)
    h, a1k, a2k = _pre_call(x, W_gat, al, ar)

    sd = jnp.transpose(edge_index.reshape(2, NW, NCH, CHUNK), (1, 2, 0, 3))
    num, den = _sc_edge(sd, a1k.reshape(N), a2k.reshape(N), h)

    we = jnp.concatenate([We1, We2, We3], axis=1)
    be = jnp.concatenate([be1, be2, be3]).reshape(1, 3 * EMB)
    cat, tp = _post_call(
        num, den.reshape(NC, NP, 1), h_prev1, h_prev2, b_gat.reshape(1, HID),
        Wz1, bz1.reshape(1, HID), Wh1, bh1.reshape(1, HID),
        Wz2, bz2.reshape(1, HID), Wh2, bh2.reshape(1, HID),
        we, be, Wf, bf.reshape(1, 1))
    return (cat.reshape(N, 3, EMB), tp.reshape(N))


# speedup vs baseline: 39.0873x; 1.0102x over previous
import functools

import jax
import jax.numpy as jnp
from jax import lax
from jax.experimental import pallas as pl
from jax.experimental.pallas import tpu as pltpu
from jax.experimental.pallas import tpu_sc as plsc

N = 10000
E = 320000
D_IN = 128
HID = 128
EMB = 16

NC = 2
NS = 16
NW = NC * NS
EPW = E // NW
CHUNK = 80
NCH = EPW // CHUNK
NP = 10240
RPT = NP // NS
DCH = 2048


def _pre_body(x_ref, wg_ref, al_ref, ar_ref, h_ref, a1_ref, a2_ref):
    h = jnp.dot(x_ref[...], wg_ref[...], preferred_element_type=jnp.float32)
    h_ref[...] = h
    a1_ref[...] = jnp.sum(h * al_ref[...], axis=1, keepdims=True)
    a2_ref[...] = jnp.sum(h * ar_ref[...], axis=1, keepdims=True)


def _pre_call(x, wg, al, ar):
    return pl.pallas_call(
        _pre_body,
        out_shape=[
            jax.ShapeDtypeStruct((N, HID), jnp.float32),
            jax.ShapeDtypeStruct((N, 1), jnp.float32),
            jax.ShapeDtypeStruct((N, 1), jnp.float32),
        ],
    )(x, wg, al, ar)


_sc_mesh = plsc.VectorSubcoreMesh(core_axis_name="c", subcore_axis_name="s")


@functools.partial(
    pl.kernel,
    out_type=(
        jax.ShapeDtypeStruct((NC, NP, HID), jnp.float32),
        jax.ShapeDtypeStruct((NC, NP), jnp.float32),
    ),
    mesh=_sc_mesh,
    scratch_types=[
        pltpu.VMEM((NP,), jnp.float32),
        pltpu.VMEM((N,), jnp.float32),
        pltpu.VMEM((2, CHUNK), jnp.int32),
        pltpu.VMEM((2, CHUNK), jnp.int32),
        pltpu.VMEM((1, CHUNK), jnp.float32),
        pltpu.VMEM((CHUNK, HID), jnp.float32),
        pltpu.VMEM((CHUNK, HID), jnp.float32),
        pltpu.VMEM_SHARED((NP, HID), jnp.float32),
        pltpu.VMEM_SHARED((NP,), jnp.float32),
        pltpu.SemaphoreType.DMA,
        pltpu.SemaphoreType.DMA,
        pltpu.SemaphoreType.DMA,
        pltpu.SemaphoreType.DMA,
    ],
    compiler_params=pltpu.CompilerParams(needs_layout_passes=False),
)
def _sc_edge(sd_hbm, a1_hbm, a2_hbm, h_hbm, num_out, den_out,
             a1_v, a2_v, sdc0, sdc1, wc_v, rows0, rows1,
             num_sh, den_sh, gsem0, gsem1, isem0, isem1):
    c = lax.axis_index("c")
    s = lax.axis_index("s")
    wid = c * NS + s
    zf = jnp.zeros((16,), jnp.float32)

    def _zrow(k, _):
        for j in range(HID // 16):
            rows0[k, pl.ds(j * 16, 16)] = zf
        return 0

    lax.fori_loop(0, CHUNK, _zrow, 0)

    def _za(k, _):
        a1_v[pl.ds(k * 16, 16)] = zf
        return 0

    lax.fori_loop(0, NP // 16, _za, 0)

    for t in range(RPT // CHUNK):
        pltpu.sync_copy(rows0, num_sh.at[pl.ds(s * RPT + t * CHUNK, CHUNK)])

    @pl.when(s < NP // DCH)
    def _():
        pltpu.sync_copy(a1_v.at[pl.ds(0, DCH)], den_sh.at[pl.ds(s * DCH, DCH)])

    pltpu.sync_copy(a1_hbm, a1_v.at[pl.ds(0, N)])
    pltpu.sync_copy(a2_hbm, a2_v)

    bufs = ((sdc0, rows0, gsem0, isem0), (sdc1, rows1, gsem1, isem1))

    def _idx_start(ii, b):
        sdc, _, _, isem = bufs[b]
        pltpu.async_copy(sd_hbm.at[wid, ii], sdc, isem)

    def _idx_wait(b):
        sdc, _, _, isem = bufs[b]
        pltpu.make_async_copy(sd_hbm.at[wid, 0], sdc, isem).wait()

    def _gather_start(b):
        sdc, rows, gsem, _ = bufs[b]
        pltpu.async_copy(h_hbm.at[sdc.at[0]], rows, gsem)

    def _gather_wait(b):
        sdc, rows, gsem, _ = bufs[b]
        pltpu.make_async_copy(h_hbm.at[sdc.at[0]], rows, gsem).wait()

    def _process(ii, b, steady):
        sdc, rows, _, _ = bufs[b]
        nb = 1 - b
        if steady:
            _idx_wait(nb)
            _gather_start(nb)

        for j in range(CHUNK // 16):
            s16 = sdc[0, pl.ds(j * 16, 16)]
            d16 = sdc[1, pl.ds(j * 16, 16)]
            e = plsc.load_gather(a1_v, [s16]) + plsc.load_gather(a2_v, [d16])
            e = jnp.where(e >= 0, e, 0.2 * e)
            wc_v[0, pl.ds(j * 16, 16)] = jnp.exp(e)

        _gather_wait(b)

        @plsc.parallel_loop(0, CHUNK // 16)
        def _scale(g):
            w16 = wc_v[0, pl.ds(g * 16, 16)]
            base = g * 16
            for t in range(16):
                wk = jnp.full((16,), w16[t])
                for j in range(HID // 16):
                    rows[base + t, pl.ds(j * 16, 16)] = (
                        rows[base + t, pl.ds(j * 16, 16)] * wk)

        pltpu.sync_copy(rows, num_sh.at[sdc.at[1]], add=True)
        pltpu.sync_copy(wc_v.at[0], den_sh.at[sdc.at[1]], add=True)

        if steady:
            @pl.when(ii < NCH - 2)
            def _():
                _idx_start(ii + 2, b)

    _idx_start(0, 0)
    _idx_wait(0)
    _gather_start(0)
    _idx_start(1, 1)

    plsc.subcore_barrier()

    def _pair(t, _):
        _process(2 * t, 0, True)
        _process(2 * t + 1, 1, True)
        return 0

    lax.fori_loop(0, (NCH - 1) // 2, _pair, 0)
    _process(NCH - 1, 0, False)

    plsc.subcore_barrier()

    pltpu.sync_copy(num_sh.at[pl.ds(s * RPT, RPT)],
                    num_out.at[c, pl.ds(s * RPT, RPT)])

    @pl.when(s < NP // DCH)
    def _():
        pltpu.sync_copy(den_sh.at[pl.ds(s * DCH, DCH)],
                        den_out.at[c, pl.ds(s * DCH, DCH)])


BLK = 1000


def _post_body(num_ref, den_ref, hp1_ref, hp2_ref, bg_ref,
               wz1_ref, bz1_ref, wh1_ref, bh1_ref,
               wz2_ref, bz2_ref, wh2_ref, bh2_ref,
               we_ref, be_ref, wf_ref, bf_ref, c_ref, tp_ref):
    num = num_ref[0] + num_ref[1]
    den = den_ref[0, :, 0] + den_ref[1, :, 0]
    out = num / (den[:, None] + 1e-16) + bg_ref[...]
    z1 = jax.nn.sigmoid(
        jnp.dot(out, wz1_ref[...], preferred_element_type=jnp.float32)
        + bz1_ref[...])
    ht1 = jnp.tanh(
        jnp.dot(out, wh1_ref[...], preferred_element_type=jnp.float32)
        + bh1_ref[...])
    h1 = (1.0 - z1) * hp1_ref[...] + z1 * ht1
    z2 = jax.nn.sigmoid(
        jnp.dot(h1, wz2_ref[...], preferred_element_type=jnp.float32)
        + bz2_ref[...])
    ht2 = jnp.tanh(
        jnp.dot(h1, wh2_ref[...], preferred_element_type=jnp.float32)
        + bh2_ref[...])
    h2 = (1.0 - z2) * hp2_ref[...] + z2 * ht2
    c_ref[...] = (jnp.dot(h2, we_ref[...], preferred_element_type=jnp.float32)
                  + be_ref[...])
    tp_ref[...] = (jnp.dot(h2, wf_ref[...], preferred_element_type=jnp.float32)
                   + bf_ref[...])


def _post_call(num, den3, hp1, hp2, bg, wz1, bz1, wh1, bh1,
               wz2, bz2, wh2, bh2, we, be, wf, bf):
    full = lambda shape: pl.BlockSpec(shape, lambda i: (0,) * len(shape))
    return pl.pallas_call(
        _post_body,
        grid=(N // BLK,),
        in_specs=[
            pl.BlockSpec((NC, BLK, HID), lambda i: (0, i, 0)),
            pl.BlockSpec((NC, BLK, 1), lambda i: (0, i, 0)),
            pl.BlockSpec((BLK, HID), lambda i: (i, 0)),
            pl.BlockSpec((BLK, HID), lambda i: (i, 0)),
            full((1, HID)),
            full((HID, HID)), full((1, HID)),
            full((HID, HID)), full((1, HID)),
            full((HID, HID)), full((1, HID)),
            full((HID, HID)), full((1, HID)),
            full((HID, 3 * EMB)), full((1, 3 * EMB)),
            full((HID, 1)), full((1, 1)),
        ],
        out_specs=[
            pl.BlockSpec((BLK, 3 * EMB), lambda i: (i, 0)),
            pl.BlockSpec((BLK, 1), lambda i: (i, 0)),
        ],
        out_shape=[
            jax.ShapeDtypeStruct((N, 3 * EMB), jnp.float32),
            jax.ShapeDtypeStruct((N, 1), jnp.float32),
        ],
    )(num, den3, hp1, hp2, bg, wz1, bz1, wh1, bh1,
      wz2, bz2, wh2, bh2, we, be, wf, bf)


def kernel(x, edge_index, h_prev1, h_prev2, W_gat, attn_l, attn_r, b_gat,
           Wz1, bz1, Wh1, bh1, Wz2, bz2, Wh2, bh2,
           We1, be1, We2, be2, We3, be3, Wf, bf):
    al = attn_l.reshape(1, HID)
    ar = attn_r.reshape(1, HID)
    h, a1k, a2k = _pre_call(x, W_gat, al, ar)

    sd = jnp.transpose(edge_index.reshape(2, NW, NCH, CHUNK), (1, 2, 0, 3))
    num, den = _sc_edge(sd, a1k.reshape(N), a2k.reshape(N), h)

    we = jnp.concatenate([We1, We2, We3], axis=1)
    be = jnp.concatenate([be1, be2, be3]).reshape(1, 3 * EMB)
    cat, tp = _post_call(
        num, den.reshape(NC, NP, 1), h_prev1, h_prev2, b_gat.reshape(1, HID),
        Wz1, bz1.reshape(1, HID), Wh1, bh1.reshape(1, HID),
        Wz2, bz2.reshape(1, HID), Wh2, bh2.reshape(1, HID),
        we, be, Wf, bf.reshape(1, 1))
    return (cat.reshape(N, 3, EMB), tp.reshape(N))


# speedup vs baseline: 47.8296x; 1.2237x over previous
import functools

import jax
import jax.numpy as jnp
from jax import lax
from jax.experimental import pallas as pl
from jax.experimental.pallas import tpu as pltpu
from jax.experimental.pallas import tpu_sc as plsc

N = 10000
E = 320000
D_IN = 128
HID = 128
EMB = 16

NC = 2
NS = 16
NW = NC * NS
EPW = E // NW
CHUNK = 80
NCH = EPW // CHUNK
NP = 10240
RPT = NP // NS
DCH = 2048


def _pre_body(x_ref, wg_ref, al_ref, ar_ref, h_ref, a1_ref, a2_ref):
    h = jnp.dot(x_ref[...], wg_ref[...], preferred_element_type=jnp.float32)
    h_ref[...] = h
    a1_ref[...] = jnp.sum(h * al_ref[...], axis=1, keepdims=True)
    a2_ref[...] = jnp.sum(h * ar_ref[...], axis=1, keepdims=True)


def _pre_call(x, wg, al, ar):
    return pl.pallas_call(
        _pre_body,
        out_shape=[
            jax.ShapeDtypeStruct((N, HID), jnp.float32),
            jax.ShapeDtypeStruct((N, 1), jnp.float32),
            jax.ShapeDtypeStruct((N, 1), jnp.float32),
        ],
    )(x, wg, al, ar)


_sc_mesh = plsc.VectorSubcoreMesh(core_axis_name="c", subcore_axis_name="s")


@functools.partial(
    pl.kernel,
    out_type=(
        jax.ShapeDtypeStruct((NC, NP, HID), jnp.float32),
        jax.ShapeDtypeStruct((NC, NP), jnp.float32),
    ),
    mesh=_sc_mesh,
    scratch_types=[
        pltpu.VMEM((NP,), jnp.float32),
        pltpu.VMEM((N,), jnp.float32),
        pltpu.VMEM((2, CHUNK), jnp.int32),
        pltpu.VMEM((2, CHUNK), jnp.int32),
        pltpu.VMEM((1, CHUNK), jnp.int32),
        pltpu.VMEM((1, CHUNK), jnp.int32),
        pltpu.VMEM((1, CHUNK), jnp.float32),
        pltpu.VMEM((1, CHUNK), jnp.float32),
        pltpu.VMEM((CHUNK, HID), jnp.float32),
        pltpu.VMEM((CHUNK, HID), jnp.float32),
        pltpu.VMEM_SHARED((NP, HID), jnp.float32),
        pltpu.VMEM_SHARED((NP,), jnp.float32),
        pltpu.SemaphoreType.DMA,
        pltpu.SemaphoreType.DMA,
        pltpu.SemaphoreType.DMA,
        pltpu.SemaphoreType.DMA,
        pltpu.SemaphoreType.DMA,
        pltpu.SemaphoreType.DMA,
        pltpu.SemaphoreType.DMA,
        pltpu.SemaphoreType.DMA,
    ],
    compiler_params=pltpu.CompilerParams(needs_layout_passes=False),
)
def _sc_edge(sd_hbm, a1_hbm, a2_hbm, h_hbm, num_out, den_out,
             a1_v, a2_v, sdc0, sdc1, dstx0, dstx1, wc0, wc1, rows0, rows1,
             num_sh, den_sh, gsem0, gsem1, isem0, isem1,
             nsem0, nsem1, dsem0, dsem1):
    c = lax.axis_index("c")
    s = lax.axis_index("s")
    wid = c * NS + s
    zf = jnp.zeros((16,), jnp.float32)

    def _zrow(k, _):
        for j in range(HID // 16):
            rows0[k, pl.ds(j * 16, 16)] = zf
        return 0

    lax.fori_loop(0, CHUNK, _zrow, 0)

    def _za(k, _):
        a1_v[pl.ds(k * 16, 16)] = zf
        return 0

    lax.fori_loop(0, NP // 16, _za, 0)

    for t in range(RPT // CHUNK):
        pltpu.sync_copy(rows0, num_sh.at[pl.ds(s * RPT + t * CHUNK, CHUNK)])

    @pl.when(s < NP // DCH)
    def _():
        pltpu.sync_copy(a1_v.at[pl.ds(0, DCH)], den_sh.at[pl.ds(s * DCH, DCH)])

    pltpu.sync_copy(a1_hbm, a1_v.at[pl.ds(0, N)])
    pltpu.sync_copy(a2_hbm, a2_v)

    bufs = ((sdc0, rows0, dstx0, wc0, gsem0, isem0, nsem0, dsem0),
            (sdc1, rows1, dstx1, wc1, gsem1, isem1, nsem1, dsem1))

    def _idx_start(ii, b):
        sdc, _, _, _, _, isem, _, _ = bufs[b]
        pltpu.async_copy(sd_hbm.at[wid, ii], sdc, isem)

    def _idx_wait(b):
        sdc, _, _, _, _, isem, _, _ = bufs[b]
        pltpu.make_async_copy(sd_hbm.at[wid, 0], sdc, isem).wait()

    def _gather_start(b):
        sdc, rows, _, _, gsem, _, _, _ = bufs[b]
        pltpu.async_copy(h_hbm.at[sdc.at[0]], rows, gsem)

    def _gather_wait(b):
        sdc, rows, _, _, gsem, _, _, _ = bufs[b]
        pltpu.make_async_copy(h_hbm.at[sdc.at[0]], rows, gsem).wait()

    def _nscat_wait(b):
        _, rows, dstx, _, _, _, nsem, _ = bufs[b]
        pltpu.make_async_copy(rows, num_sh.at[dstx.at[0]], nsem).wait()

    def _dscat_wait(b):
        _, _, dstx, wc, _, _, _, dsem = bufs[b]
        pltpu.make_async_copy(wc.at[0], den_sh.at[dstx.at[0]], dsem).wait()

    def _process(ii, b, steady):
        sdc, rows, dstx, wc, _, _, nsem, dsem = bufs[b]
        nb = 1 - b
        if steady:
            _idx_wait(nb)

            @pl.when(ii > 0)
            def _():
                _nscat_wait(nb)

            _gather_start(nb)

        @pl.when(ii > 1)
        def _():
            _dscat_wait(b)

        for j in range(CHUNK // 16):
            s16 = sdc[0, pl.ds(j * 16, 16)]
            d16 = sdc[1, pl.ds(j * 16, 16)]
            e = plsc.load_gather(a1_v, [s16]) + plsc.load_gather(a2_v, [d16])
            e = jnp.where(e >= 0, e, 0.2 * e)
            wc[0, pl.ds(j * 16, 16)] = jnp.exp(e)

        _gather_wait(b)

        for j in range(CHUNK // 16):
            dstx[0, pl.ds(j * 16, 16)] = sdc[1, pl.ds(j * 16, 16)]

        @plsc.parallel_loop(0, CHUNK // 16)
        def _scale(g):
            w16 = wc[0, pl.ds(g * 16, 16)]
            base = g * 16
            for t in range(16):
                wk = jnp.full((16,), w16[t])
                for j in range(HID // 16):
                    rows[base + t, pl.ds(j * 16, 16)] = (
                        rows[base + t, pl.ds(j * 16, 16)] * wk)

        pltpu.async_copy(rows, num_sh.at[dstx.at[0]], nsem, add=True)
        pltpu.async_copy(wc.at[0], den_sh.at[dstx.at[0]], dsem, add=True)

        if steady:
            @pl.when(ii < NCH - 2)
            def _():
                _idx_start(ii + 2, b)

    _idx_start(0, 0)
    _idx_wait(0)
    _gather_start(0)
    _idx_start(1, 1)

    plsc.subcore_barrier()

    def _pair(t, _):
        _process(2 * t, 0, True)
        _process(2 * t + 1, 1, True)
        return 0

    lax.fori_loop(0, (NCH - 1) // 2, _pair, 0)
    _process(NCH - 1, 0, False)

    _nscat_wait(1)
    _nscat_wait(0)
    _dscat_wait(1)
    _dscat_wait(0)

    plsc.subcore_barrier()

    pltpu.sync_copy(num_sh.at[pl.ds(s * RPT, RPT)],
                    num_out.at[c, pl.ds(s * RPT, RPT)])

    @pl.when(s < NP // DCH)
    def _():
        pltpu.sync_copy(den_sh.at[pl.ds(s * DCH, DCH)],
                        den_out.at[c, pl.ds(s * DCH, DCH)])


BLK = 1000


def _post_body(num_ref, den_ref, hp1_ref, hp2_ref, bg_ref,
               wz1_ref, bz1_ref, wh1_ref, bh1_ref,
               wz2_ref, bz2_ref, wh2_ref, bh2_ref,
               we_ref, be_ref, wf_ref, bf_ref, c_ref, tp_ref):
    num = num_ref[0] + num_ref[1]
    den = den_ref[0, :, 0] + den_ref[1, :, 0]
    out = num / (den[:, None] + 1e-16) + bg_ref[...]
    z1 = jax.nn.sigmoid(
        jnp.dot(out, wz1_ref[...], preferred_element_type=jnp.float32)
        + bz1_ref[...])
    ht1 = jnp.tanh(
        jnp.dot(out, wh1_ref[...], preferred_element_type=jnp.float32)
        + bh1_ref[...])
    h1 = (1.0 - z1) * hp1_ref[...] + z1 * ht1
    z2 = jax.nn.sigmoid(
        jnp.dot(h1, wz2_ref[...], preferred_element_type=jnp.float32)
        + bz2_ref[...])
    ht2 = jnp.tanh(
        jnp.dot(h1, wh2_ref[...], preferred_element_type=jnp.float32)
        + bh2_ref[...])
    h2 = (1.0 - z2) * hp2_ref[...] + z2 * ht2
    c_ref[...] = (jnp.dot(h2, we_ref[...], preferred_element_type=jnp.float32)
                  + be_ref[...])
    tp_ref[...] = (jnp.dot(h2, wf_ref[...], preferred_element_type=jnp.float32)
                   + bf_ref[...])


def _post_call(num, den3, hp1, hp2, bg, wz1, bz1, wh1, bh1,
               wz2, bz2, wh2, bh2, we, be, wf, bf):
    full = lambda shape: pl.BlockSpec(shape, lambda i: (0,) * len(shape))
    return pl.pallas_call(
        _post_body,
        grid=(N // BLK,),
        in_specs=[
            pl.BlockSpec((NC, BLK, HID), lambda i: (0, i, 0)),
            pl.BlockSpec((NC, BLK, 1), lambda i: (0, i, 0)),
            pl.BlockSpec((BLK, HID), lambda i: (i, 0)),
            pl.BlockSpec((BLK, HID), lambda i: (i, 0)),
            full((1, HID)),
            full((HID, HID)), full((1, HID)),
            full((HID, HID)), full((1, HID)),
            full((HID, HID)), full((1, HID)),
            full((HID, HID)), full((1, HID)),
            full((HID, 3 * EMB)), full((1, 3 * EMB)),
            full((HID, 1)), full((1, 1)),
        ],
        out_specs=[
            pl.BlockSpec((BLK, 3 * EMB), lambda i: (i, 0)),
            pl.BlockSpec((BLK, 1), lambda i: (i, 0)),
        ],
        out_shape=[
            jax.ShapeDtypeStruct((N, 3 * EMB), jnp.float32),
            jax.ShapeDtypeStruct((N, 1), jnp.float32),
        ],
    )(num, den3, hp1, hp2, bg, wz1, bz1, wh1, bh1,
      wz2, bz2, wh2, bh2, we, be, wf, bf)


def kernel(x, edge_index, h_prev1, h_prev2, W_gat, attn_l, attn_r, b_gat,
           Wz1, bz1, Wh1, bh1, Wz2, bz2, Wh2, bh2,
           We1, be1, We2, be2, We3, be3, Wf, bf):
    al = attn_l.reshape(1, HID)
    ar = attn_r.reshape(1, HID)
    h, a1k, a2k = _pre_call(x, W_gat, al, ar)

    sd = jnp.transpose(edge_index.reshape(2, NW, NCH, CHUNK), (1, 2, 0, 3))
    num, den = _sc_edge(sd, a1k.reshape(N), a2k.reshape(N), h)

    we = jnp.concatenate([We1, We2, We3], axis=1)
    be = jnp.concatenate([be1, be2, be3]).reshape(1, 3 * EMB)
    cat, tp = _post_call(
        num, den.reshape(NC, NP, 1), h_prev1, h_prev2, b_gat.reshape(1, HID),
        Wz1, bz1.reshape(1, HID), Wh1, bh1.reshape(1, HID),
        Wz2, bz2.reshape(1, HID), Wh2, bh2.reshape(1, HID),
        we, be, Wf, bf.reshape(1, 1))
    return (cat.reshape(N, 3, EMB), tp.reshape(N))


# speedup vs baseline: 49.6922x; 1.0389x over previous
import functools

import jax
import jax.numpy as jnp
from jax import lax
from jax.experimental import pallas as pl
from jax.experimental.pallas import tpu as pltpu
from jax.experimental.pallas import tpu_sc as plsc

N = 10000
E = 320000
D_IN = 128
HID = 128
EMB = 16

NC = 2
NS = 16
NW = NC * NS
EPW = E // NW
CHUNK = 80
NCH = EPW // CHUNK
NP = 10240
RPT = NP // NS
DCH = 2048


def _pre_body(x_ref, wg_ref, al_ref, ar_ref, h_ref, a1_ref, a2_ref):
    h = jnp.dot(x_ref[...], wg_ref[...], preferred_element_type=jnp.float32)
    h_ref[...] = h
    a1_ref[...] = jnp.sum(h * al_ref[...], axis=1, keepdims=True)
    a2_ref[...] = jnp.sum(h * ar_ref[...], axis=1, keepdims=True)


def _pre_call(x, wg, al, ar):
    return pl.pallas_call(
        _pre_body,
        out_shape=[
            jax.ShapeDtypeStruct((N, HID), jnp.float32),
            jax.ShapeDtypeStruct((N, 1), jnp.float32),
            jax.ShapeDtypeStruct((N, 1), jnp.float32),
        ],
    )(x, wg, al, ar)


_sc_mesh = plsc.VectorSubcoreMesh(core_axis_name="c", subcore_axis_name="s")


@functools.partial(
    pl.kernel,
    out_type=(
        jax.ShapeDtypeStruct((NC, NP, HID), jnp.float32),
        jax.ShapeDtypeStruct((NC, NP), jnp.float32),
    ),
    mesh=_sc_mesh,
    scratch_types=[
        pltpu.VMEM((NP,), jnp.float32),
        pltpu.VMEM((N,), jnp.float32),
        pltpu.VMEM((2, CHUNK), jnp.int32),
        pltpu.VMEM((2, CHUNK), jnp.int32),
        pltpu.VMEM((1, CHUNK), jnp.int32),
        pltpu.VMEM((1, CHUNK), jnp.int32),
        pltpu.VMEM((1, CHUNK), jnp.float32),
        pltpu.VMEM((1, CHUNK), jnp.float32),
        pltpu.VMEM((CHUNK, HID), jnp.float32),
        pltpu.VMEM((CHUNK, HID), jnp.float32),
        pltpu.VMEM_SHARED((NP, HID), jnp.float32),
        pltpu.VMEM_SHARED((NP,), jnp.float32),
        pltpu.SemaphoreType.DMA,
        pltpu.SemaphoreType.DMA,
        pltpu.SemaphoreType.DMA,
        pltpu.SemaphoreType.DMA,
        pltpu.SemaphoreType.DMA,
        pltpu.SemaphoreType.DMA,
        pltpu.SemaphoreType.DMA,
        pltpu.SemaphoreType.DMA,
    ],
    compiler_params=pltpu.CompilerParams(needs_layout_passes=False),
)
def _sc_edge(sd_hbm, a1_hbm, a2_hbm, h_hbm, num_out, den_out,
             a1_v, a2_v, sdc0, sdc1, dstx0, dstx1, wc0, wc1, rows0, rows1,
             num_sh, den_sh, gsem0, gsem1, isem0, isem1,
             nsem0, nsem1, dsem0, dsem1):
    c = lax.axis_index("c")
    s = lax.axis_index("s")
    wid = c * NS + s
    zf = jnp.zeros((16,), jnp.float32)

    def _zrow(k, _):
        for j in range(HID // 16):
            rows0[k, pl.ds(j * 16, 16)] = zf
        return 0

    lax.fori_loop(0, CHUNK, _zrow, 0)

    def _za(k, _):
        a1_v[pl.ds(k * 16, 16)] = zf
        return 0

    lax.fori_loop(0, NP // 16, _za, 0)

    for t in range(RPT // CHUNK):
        pltpu.sync_copy(rows0, num_sh.at[pl.ds(s * RPT + t * CHUNK, CHUNK)])

    @pl.when(s < NP // DCH)
    def _():
        pltpu.sync_copy(a1_v.at[pl.ds(0, DCH)], den_sh.at[pl.ds(s * DCH, DCH)])

    pltpu.sync_copy(a1_hbm, a1_v.at[pl.ds(0, N)])
    pltpu.sync_copy(a2_hbm, a2_v)

    bufs = ((sdc0, rows0, dstx0, wc0, gsem0, isem0, nsem0, dsem0),
            (sdc1, rows1, dstx1, wc1, gsem1, isem1, nsem1, dsem1))

    ebase = wid * EPW

    def _idx_start(ii, b):
        sdc, _, _, _, _, isem, _, _ = bufs[b]
        pltpu.async_copy(sd_hbm.at[pl.ds(ebase + ii * CHUNK, CHUNK)],
                         sdc.at[0], isem)
        pltpu.async_copy(sd_hbm.at[pl.ds(E + ebase + ii * CHUNK, CHUNK)],
                         sdc.at[1], isem)

    def _idx_wait(b):
        sdc, _, _, _, _, isem, _, _ = bufs[b]
        pltpu.make_async_copy(sd_hbm.at[pl.ds(0, CHUNK)], sdc.at[0], isem).wait()
        pltpu.make_async_copy(sd_hbm.at[pl.ds(0, CHUNK)], sdc.at[1], isem).wait()

    def _gather_start(b):
        sdc, rows, _, _, gsem, _, _, _ = bufs[b]
        pltpu.async_copy(h_hbm.at[sdc.at[0]], rows, gsem)

    def _gather_wait(b):
        sdc, rows, _, _, gsem, _, _, _ = bufs[b]
        pltpu.make_async_copy(h_hbm.at[sdc.at[0]], rows, gsem).wait()

    def _nscat_wait(b):
        _, rows, dstx, _, _, _, nsem, _ = bufs[b]
        pltpu.make_async_copy(rows, num_sh.at[dstx.at[0]], nsem).wait()

    def _dscat_wait(b):
        _, _, dstx, wc, _, _, _, dsem = bufs[b]
        pltpu.make_async_copy(wc.at[0], den_sh.at[dstx.at[0]], dsem).wait()

    def _process(ii, b, steady):
        sdc, rows, dstx, wc, _, _, nsem, dsem = bufs[b]
        nb = 1 - b
        if steady:
            _idx_wait(nb)

            @pl.when(ii > 0)
            def _():
                _nscat_wait(nb)

            _gather_start(nb)

        @pl.when(ii > 1)
        def _():
            _dscat_wait(b)

        for j in range(CHUNK // 16):
            s16 = sdc[0, pl.ds(j * 16, 16)]
            d16 = sdc[1, pl.ds(j * 16, 16)]
            e = plsc.load_gather(a1_v, [s16]) + plsc.load_gather(a2_v, [d16])
            e = jnp.where(e >= 0, e, 0.2 * e)
            wc[0, pl.ds(j * 16, 16)] = jnp.exp(e)

        _gather_wait(b)

        for j in range(CHUNK // 16):
            dstx[0, pl.ds(j * 16, 16)] = sdc[1, pl.ds(j * 16, 16)]

        @plsc.parallel_loop(0, CHUNK // 16, unroll=2)
        def _scale(g):
            w16 = wc[0, pl.ds(g * 16, 16)]
            base = g * 16
            for t in range(16):
                wk = jnp.full((16,), w16[t])
                for j in range(HID // 16):
                    rows[base + t, pl.ds(j * 16, 16)] = (
                        rows[base + t, pl.ds(j * 16, 16)] * wk)

        pltpu.async_copy(rows, num_sh.at[dstx.at[0]], nsem, add=True)
        pltpu.async_copy(wc.at[0], den_sh.at[dstx.at[0]], dsem, add=True)

        if steady:
            @pl.when(ii < NCH - 2)
            def _():
                _idx_start(ii + 2, b)

    _idx_start(0, 0)
    _idx_wait(0)
    _gather_start(0)
    _idx_start(1, 1)

    plsc.subcore_barrier()

    def _pair(t, _):
        _process(2 * t, 0, True)
        _process(2 * t + 1, 1, True)
        return 0

    lax.fori_loop(0, (NCH - 1) // 2, _pair, 0)
    _process(NCH - 1, 0, False)

    _nscat_wait(1)
    _nscat_wait(0)
    _dscat_wait(1)
    _dscat_wait(0)

    plsc.subcore_barrier()

    pltpu.sync_copy(num_sh.at[pl.ds(s * RPT, RPT)],
                    num_out.at[c, pl.ds(s * RPT, RPT)])

    @pl.when(s < NP // DCH)
    def _():
        pltpu.sync_copy(den_sh.at[pl.ds(s * DCH, DCH)],
                        den_out.at[c, pl.ds(s * DCH, DCH)])


BLK = 1000


def _post_body(num_ref, den_ref, hp1_ref, hp2_ref, bg_ref,
               wz1_ref, bz1_ref, wh1_ref, bh1_ref,
               wz2_ref, bz2_ref, wh2_ref, bh2_ref,
               we_ref, be_ref, wf_ref, bf_ref, c_ref, tp_ref):
    num = num_ref[0] + num_ref[1]
    den = den_ref[0, :, 0] + den_ref[1, :, 0]
    out = num / (den[:, None] + 1e-16) + bg_ref[...]
    z1 = jax.nn.sigmoid(
        jnp.dot(out, wz1_ref[...], preferred_element_type=jnp.float32)
        + bz1_ref[...])
    ht1 = jnp.tanh(
        jnp.dot(out, wh1_ref[...], preferred_element_type=jnp.float32)
        + bh1_ref[...])
    h1 = (1.0 - z1) * hp1_ref[...] + z1 * ht1
    z2 = jax.nn.sigmoid(
        jnp.dot(h1, wz2_ref[...], preferred_element_type=jnp.float32)
        + bz2_ref[...])
    ht2 = jnp.tanh(
        jnp.dot(h1, wh2_ref[...], preferred_element_type=jnp.float32)
        + bh2_ref[...])
    h2 = (1.0 - z2) * hp2_ref[...] + z2 * ht2
    c_ref[...] = (jnp.dot(h2, we_ref[...], preferred_element_type=jnp.float32)
                  + be_ref[...])
    tp_ref[...] = (jnp.dot(h2, wf_ref[...], preferred_element_type=jnp.float32)
                   + bf_ref[...])


def _post_call(num, den3, hp1, hp2, bg, wz1, bz1, wh1, bh1,
               wz2, bz2, wh2, bh2, we, be, wf, bf):
    full = lambda shape: pl.BlockSpec(shape, lambda i: (0,) * len(shape))
    return pl.pallas_call(
        _post_body,
        grid=(N // BLK,),
        in_specs=[
            pl.BlockSpec((NC, BLK, HID), lambda i: (0, i, 0)),
            pl.BlockSpec((NC, BLK, 1), lambda i: (0, i, 0)),
            pl.BlockSpec((BLK, HID), lambda i: (i, 0)),
            pl.BlockSpec((BLK, HID), lambda i: (i, 0)),
            full((1, HID)),
            full((HID, HID)), full((1, HID)),
            full((HID, HID)), full((1, HID)),
            full((HID, HID)), full((1, HID)),
            full((HID, HID)), full((1, HID)),
            full((HID, 3 * EMB)), full((1, 3 * EMB)),
            full((HID, 1)), full((1, 1)),
        ],
        out_specs=[
            pl.BlockSpec((BLK, 3 * EMB), lambda i: (i, 0)),
            pl.BlockSpec((BLK, 1), lambda i: (i, 0)),
        ],
        out_shape=[
            jax.ShapeDtypeStruct((N, 3 * EMB), jnp.float32),
            jax.ShapeDtypeStruct((N, 1), jnp.float32),
        ],
    )(num, den3, hp1, hp2, bg, wz1, bz1, wh1, bh1,
      wz2, bz2, wh2, bh2, we, be, wf, bf)


def kernel(x, edge_index, h_prev1, h_prev2, W_gat, attn_l, attn_r, b_gat,
           Wz1, bz1, Wh1, bh1, Wz2, bz2, Wh2, bh2,
           We1, be1, We2, be2, We3, be3, Wf, bf):
    al = attn_l.reshape(1, HID)
    ar = attn_r.reshape(1, HID)
    h, a1k, a2k = _pre_call(x, W_gat, al, ar)

    num, den = _sc_edge(edge_index.reshape(2 * E), a1k.reshape(N),
                        a2k.reshape(N), h)

    we = jnp.concatenate([We1, We2, We3], axis=1)
    be = jnp.concatenate([be1, be2, be3]).reshape(1, 3 * EMB)
    cat, tp = _post_call(
        num, den.reshape(NC, NP, 1), h_prev1, h_prev2, b_gat.reshape(1, HID),
        Wz1, bz1.reshape(1, HID), Wh1, bh1.reshape(1, HID),
        Wz2, bz2.reshape(1, HID), Wh2, bh2.reshape(1, HID),
        we, be, Wf, bf.reshape(1, 1))
    return (cat.reshape(N, 3, EMB), tp.reshape(N))


# speedup vs baseline: 50.2560x; 1.0113x over previous
import functools

import jax
import jax.numpy as jnp
from jax import lax
from jax.experimental import pallas as pl
from jax.experimental.pallas import tpu as pltpu
from jax.experimental.pallas import tpu_sc as plsc

N = 10000
E = 320000
D_IN = 128
HID = 128
EMB = 16

NC = 2
NS = 16
NW = NC * NS
EPW = E // NW
CHUNK = 80
NCH = EPW // CHUNK
NP = 10240
RPT = NP // NS
DCH = 2048


def _pre_body(x_ref, wg_ref, al_ref, ar_ref, h_ref, a1_ref, a2_ref):
    h = jnp.dot(x_ref[...], wg_ref[...], preferred_element_type=jnp.float32)
    h_ref[...] = h
    a1_ref[...] = jnp.sum(h * al_ref[...], axis=1, keepdims=True)
    a2_ref[...] = jnp.sum(h * ar_ref[...], axis=1, keepdims=True)


def _pre_call(x, wg, al, ar):
    return pl.pallas_call(
        _pre_body,
        out_shape=[
            jax.ShapeDtypeStruct((N, HID), jnp.float32),
            jax.ShapeDtypeStruct((N, 1), jnp.float32),
            jax.ShapeDtypeStruct((N, 1), jnp.float32),
        ],
    )(x, wg, al, ar)


_sc_mesh = plsc.VectorSubcoreMesh(core_axis_name="c", subcore_axis_name="s")


@functools.partial(
    pl.kernel,
    out_type=(
        jax.ShapeDtypeStruct((NC, NP, HID), jnp.float32),
        jax.ShapeDtypeStruct((NC, NP), jnp.float32),
    ),
    mesh=_sc_mesh,
    scratch_types=[
        pltpu.VMEM((NP,), jnp.float32),
        pltpu.VMEM((N,), jnp.float32),
        pltpu.VMEM((2, CHUNK), jnp.int32),
        pltpu.VMEM((2, CHUNK), jnp.int32),
        pltpu.VMEM((1, CHUNK), jnp.int32),
        pltpu.VMEM((1, CHUNK), jnp.int32),
        pltpu.VMEM((1, CHUNK), jnp.float32),
        pltpu.VMEM((1, CHUNK), jnp.float32),
        pltpu.VMEM((CHUNK, HID), jnp.float32),
        pltpu.VMEM((CHUNK, HID), jnp.float32),
        pltpu.VMEM_SHARED((NP, HID), jnp.float32),
        pltpu.VMEM_SHARED((NP,), jnp.float32),
        pltpu.SemaphoreType.DMA,
        pltpu.SemaphoreType.DMA,
        pltpu.SemaphoreType.DMA,
        pltpu.SemaphoreType.DMA,
        pltpu.SemaphoreType.DMA,
        pltpu.SemaphoreType.DMA,
        pltpu.SemaphoreType.DMA,
        pltpu.SemaphoreType.DMA,
    ],
    compiler_params=pltpu.CompilerParams(needs_layout_passes=False),
)
def _sc_edge(sd_hbm, a1_hbm, a2_hbm, h_hbm, num_out, den_out,
             a1_v, a2_v, sdc0, sdc1, dstx0, dstx1, wc0, wc1, rows0, rows1,
             num_sh, den_sh, gsem0, gsem1, isem0, isem1,
             nsem0, nsem1, dsem0, dsem1):
    c = lax.axis_index("c")
    s = lax.axis_index("s")
    wid = c * NS + s
    zf = jnp.zeros((16,), jnp.float32)

    def _zrow(k, _):
        for j in range(HID // 16):
            rows0[k, pl.ds(j * 16, 16)] = zf
        return 0

    lax.fori_loop(0, CHUNK, _zrow, 0)

    def _za(k, _):
        a1_v[pl.ds(k * 16, 16)] = zf
        return 0

    lax.fori_loop(0, NP // 16, _za, 0)

    for t in range(RPT // CHUNK):
        pltpu.sync_copy(rows0, num_sh.at[pl.ds(s * RPT + t * CHUNK, CHUNK)])

    @pl.when(s < NP // DCH)
    def _():
        pltpu.sync_copy(a1_v.at[pl.ds(0, DCH)], den_sh.at[pl.ds(s * DCH, DCH)])

    pltpu.sync_copy(a1_hbm, a1_v.at[pl.ds(0, N)])
    pltpu.sync_copy(a2_hbm, a2_v)

    bufs = ((sdc0, rows0, dstx0, wc0, gsem0, isem0, nsem0, dsem0),
            (sdc1, rows1, dstx1, wc1, gsem1, isem1, nsem1, dsem1))

    ebase = wid * EPW

    def _idx_start(ii, b):
        sdc, _, _, _, _, isem, _, _ = bufs[b]
        pltpu.async_copy(sd_hbm.at[pl.ds(ebase + ii * CHUNK, CHUNK)],
                         sdc.at[0], isem)
        pltpu.async_copy(sd_hbm.at[pl.ds(E + ebase + ii * CHUNK, CHUNK)],
                         sdc.at[1], isem)

    def _idx_wait(b):
        sdc, _, _, _, _, isem, _, _ = bufs[b]
        pltpu.make_async_copy(sd_hbm.at[pl.ds(0, CHUNK)], sdc.at[0], isem).wait()
        pltpu.make_async_copy(sd_hbm.at[pl.ds(0, CHUNK)], sdc.at[1], isem).wait()

    def _gather_start(b):
        sdc, rows, _, _, gsem, _, _, _ = bufs[b]
        pltpu.async_copy(h_hbm.at[sdc.at[0]], rows, gsem)

    def _gather_wait(b):
        sdc, rows, _, _, gsem, _, _, _ = bufs[b]
        pltpu.make_async_copy(h_hbm.at[sdc.at[0]], rows, gsem).wait()

    def _nscat_wait(b):
        _, rows, dstx, _, _, _, nsem, _ = bufs[b]
        pltpu.make_async_copy(rows, num_sh.at[dstx.at[0]], nsem).wait()

    def _dscat_wait(b):
        _, _, dstx, wc, _, _, _, dsem = bufs[b]
        pltpu.make_async_copy(wc.at[0], den_sh.at[dstx.at[0]], dsem).wait()

    def _process(ii, b, steady):
        sdc, rows, dstx, wc, _, _, nsem, dsem = bufs[b]
        nb = 1 - b
        if steady:
            _idx_wait(nb)

            @pl.when(ii > 0)
            def _():
                _nscat_wait(nb)

            _gather_start(nb)

        @pl.when(ii > 1)
        def _():
            _dscat_wait(b)

        for j in range(CHUNK // 16):
            s16 = sdc[0, pl.ds(j * 16, 16)]
            d16 = sdc[1, pl.ds(j * 16, 16)]
            e = plsc.load_gather(a1_v, [s16]) + plsc.load_gather(a2_v, [d16])
            e = jnp.where(e >= 0, e, 0.2 * e)
            wc[0, pl.ds(j * 16, 16)] = jnp.exp(e)

        _gather_wait(b)

        for j in range(CHUNK // 16):
            dstx[0, pl.ds(j * 16, 16)] = sdc[1, pl.ds(j * 16, 16)]

        @plsc.parallel_loop(0, CHUNK // 16, unroll=2)
        def _scale(g):
            w16 = wc[0, pl.ds(g * 16, 16)]
            base = g * 16
            for t in range(16):
                wk = jnp.full((16,), w16[t])
                for j in range(HID // 16):
                    rows[base + t, pl.ds(j * 16, 16)] = (
                        rows[base + t, pl.ds(j * 16, 16)] * wk)

        pltpu.async_copy(rows, num_sh.at[dstx.at[0]], nsem, add=True)
        pltpu.async_copy(wc.at[0], den_sh.at[dstx.at[0]], dsem, add=True)

        if steady:
            @pl.when(ii < NCH - 2)
            def _():
                _idx_start(ii + 2, b)

    _idx_start(0, 0)
    _idx_wait(0)
    _gather_start(0)
    _idx_start(1, 1)

    plsc.subcore_barrier()

    def _pair(t, _):
        _process(2 * t, 0, True)
        _process(2 * t + 1, 1, True)
        return 0

    lax.fori_loop(0, (NCH - 1) // 2, _pair, 0)
    _process(NCH - 1, 0, False)

    _nscat_wait(1)
    _nscat_wait(0)
    _dscat_wait(1)
    _dscat_wait(0)

    plsc.subcore_barrier()

    pltpu.sync_copy(num_sh.at[pl.ds(s * RPT, RPT)],
                    num_out.at[c, pl.ds(s * RPT, RPT)])

    @pl.when(s < NP // DCH)
    def _():
        pltpu.sync_copy(den_sh.at[pl.ds(s * DCH, DCH)],
                        den_out.at[c, pl.ds(s * DCH, DCH)])


BLK = 2000
EW = 3 * EMB + 1


def _post_body(num_ref, den_ref, hp1_ref, hp2_ref, bg_ref,
               wzh1_ref, bzh1_ref, wzh2_ref, bzh2_ref,
               wef_ref, bef_ref, c_ref, tp_ref):
    num = num_ref[0] + num_ref[1]
    den = den_ref[0, :, 0] + den_ref[1, :, 0]
    out = num / (den[:, None] + 1e-16) + bg_ref[...]
    zh1 = (jnp.dot(out, wzh1_ref[...], preferred_element_type=jnp.float32)
           + bzh1_ref[...])
    z1 = jax.nn.sigmoid(zh1[:, :HID])
    ht1 = jnp.tanh(zh1[:, HID:])
    h1 = (1.0 - z1) * hp1_ref[...] + z1 * ht1
    zh2 = (jnp.dot(h1, wzh2_ref[...], preferred_element_type=jnp.float32)
           + bzh2_ref[...])
    z2 = jax.nn.sigmoid(zh2[:, :HID])
    ht2 = jnp.tanh(zh2[:, HID:])
    h2 = (1.0 - z2) * hp2_ref[...] + z2 * ht2
    ef = (jnp.dot(h2, wef_ref[...], preferred_element_type=jnp.float32)
          + bef_ref[...])
    c_ref[...] = ef[:, :3 * EMB]
    tp_ref[...] = ef[:, 3 * EMB:]


def _post_call(num, den3, hp1, hp2, bg, wzh1, bzh1, wzh2, bzh2, wef, bef):
    full = lambda shape: pl.BlockSpec(shape, lambda i: (0,) * len(shape))
    return pl.pallas_call(
        _post_body,
        grid=(N // BLK,),
        in_specs=[
            pl.BlockSpec((NC, BLK, HID), lambda i: (0, i, 0)),
            pl.BlockSpec((NC, BLK, 1), lambda i: (0, i, 0)),
            pl.BlockSpec((BLK, HID), lambda i: (i, 0)),
            pl.BlockSpec((BLK, HID), lambda i: (i, 0)),
            full((1, HID)),
            full((HID, 2 * HID)), full((1, 2 * HID)),
            full((HID, 2 * HID)), full((1, 2 * HID)),
            full((HID, EW)), full((1, EW)),
        ],
        out_specs=[
            pl.BlockSpec((BLK, 3 * EMB), lambda i: (i, 0)),
            pl.BlockSpec((BLK, 1), lambda i: (i, 0)),
        ],
        out_shape=[
            jax.ShapeDtypeStruct((N, 3 * EMB), jnp.float32),
            jax.ShapeDtypeStruct((N, 1), jnp.float32),
        ],
    )(num, den3, hp1, hp2, bg, wzh1, bzh1, wzh2, bzh2, wef, bef)


def kernel(x, edge_index, h_prev1, h_prev2, W_gat, attn_l, attn_r, b_gat,
           Wz1, bz1, Wh1, bh1, Wz2, bz2, Wh2, bh2,
           We1, be1, We2, be2, We3, be3, Wf, bf):
    al = attn_l.reshape(1, HID)
    ar = attn_r.reshape(1, HID)
    h, a1k, a2k = _pre_call(x, W_gat, al, ar)

    num, den = _sc_edge(edge_index.reshape(2 * E), a1k.reshape(N),
                        a2k.reshape(N), h)

    wzh1 = jnp.concatenate([Wz1, Wh1], axis=1)
    bzh1 = jnp.concatenate([bz1, bh1]).reshape(1, 2 * HID)
    wzh2 = jnp.concatenate([Wz2, Wh2], axis=1)
    bzh2 = jnp.concatenate([bz2, bh2]).reshape(1, 2 * HID)
    wef = jnp.concatenate([We1, We2, We3, Wf], axis=1)
    bef = jnp.concatenate([be1, be2, be3, bf]).reshape(1, EW)
    cat, tp = _post_call(
        num, den.reshape(NC, NP, 1), h_prev1, h_prev2, b_gat.reshape(1, HID),
        wzh1, bzh1, wzh2, bzh2, wef, bef)
    return (cat.reshape(N, 3, EMB), tp.reshape(N))
